# baseline (speedup 1.0000x reference)
import functools

import jax
import jax.numpy as jnp
from jax import lax
from jax.experimental import pallas as pl
from jax.experimental.pallas import tpu as pltpu
from jax.experimental.pallas import tpu_sc as plsc


def _iota16():
    return lax.iota(jnp.int32, 16)


@functools.lru_cache(maxsize=None)
def _build(b_sz: int, l_sz: int, vocab: int, hidden: int):
    info = plsc.get_sparse_core_info()
    nc, ns = info.num_cores, info.num_subcores
    nw = nc * ns
    assert hidden == 64 and b_sz % (nw * 128) == 0 and l_sz % 2 == 0
    npair = vocab // 2
    nfull = vocab // 128
    tail_w = vocab - nfull * 128
    n_tail_pair = tail_w // 2
    cpw, cextra = divmod(nfull, nw)
    assert cpw % 2 == 0

    mesh = plsc.VectorSubcoreMesh(core_axis_name="c", subcore_axis_name="s")

    @functools.partial(
        pl.kernel,
        mesh=mesh,
        compiler_params=pltpu.CompilerParams(needs_layout_passes=False),
        out_type=jax.ShapeDtypeStruct((npair, 128), jnp.float32),
        scratch_types=[
            pltpu.VMEM((hidden, 128), jnp.float32),
            pltpu.VMEM((hidden, 128), jnp.float32),
            pltpu.VMEM((64, 128), jnp.float32),
            pltpu.VMEM((64, 128), jnp.float32),
            pltpu.SemaphoreType.DMA,
            pltpu.SemaphoreType.DMA,
            pltpu.SemaphoreType.DMA,
            pltpu.SemaphoreType.DMA,
        ],
    )
    def pack_kernel(tt_hbm, tail_hbm, pairs_hbm, chunk0, chunk1,
                    pack0, pack1, rsem0, rsem1, wsem0, wsem1):
        wid = lax.axis_index("s") * nc + lax.axis_index("c")
        c0 = wid * cpw
        chunk = (chunk0, chunk1)
        pack = (pack0, pack1)
        rsem = (rsem0, rsem1)
        wsem = (wsem0, wsem1)
        hvec = [_iota16() + (k0 * 16) for k0 in range(4)]

        def fire_read(c, b):
            pltpu.async_copy(tt_hbm.at[:, pl.ds(c * 128, 128)], chunk[b],
                             rsem[b])

        def wait_read(b):
            pltpu.make_async_copy(tt_hbm.at[:, pl.ds(0, 128)], chunk[b],
                                  rsem[b]).wait()

        def fire_write(c, b):
            pltpu.async_copy(pack[b], pairs_hbm.at[pl.ds(c * 64, 64), :],
                             wsem[b])

        def wait_write(b):
            pltpu.make_async_copy(pack[b], pairs_hbm.at[pl.ds(0, 64), :],
                                  wsem[b]).wait()

        def transpose(b):
            src = chunk[b]
            dst = pack[b]

            @plsc.parallel_loop(0, 64, unroll=4)
            def prow(j):
                v0 = jnp.full((16,), 2 * j, dtype=jnp.int32)
                v1 = v0 + 1
                for k in range(4):
                    dst[j, pl.ds(k * 16, 16)] = plsc.load_gather(
                        src, [hvec[k], v0])
                for k in range(4):
                    dst[j, pl.ds(64 + k * 16, 16)] = plsc.load_gather(
                        src, [hvec[k], v1])

        def do_chunk(k, b, first, last):
            if not first:
                wait_write(b)
            if not last:
                fire_read(c0 + k + 1, 1 - b)
            wait_read(b)
            transpose(b)
            fire_write(c0 + k, b)

        fire_read(c0, 0)
        do_chunk(0, 0, True, False)
        do_chunk(1, 1, True, False)

        def step(s, carry):
            do_chunk(s * 2, 0, False, False)
            do_chunk(s * 2 + 1, 1, False, False)
            return carry

        lax.fori_loop(1, cpw // 2 - 1, step, 0)
        do_chunk(cpw - 2, 0, False, False)
        do_chunk(cpw - 1, 1, False, True)
        wait_write(0)
        wait_write(1)

        if cextra:
            @pl.when(wid < cextra)
            def _():
                c = nw * cpw + wid
                fire_read(c, 0)
                wait_read(0)
                transpose(0)
                fire_write(c, 0)
                wait_write(0)

        if n_tail_pair:
            @pl.when(wid == nw - 1)
            def _():
                pltpu.sync_copy(tail_hbm, pack0.at[pl.ds(0, n_tail_pair), :])
                pltpu.sync_copy(pack0.at[pl.ds(0, n_tail_pair), :],
                                pairs_hbm.at[pl.ds(npair - n_tail_pair,
                                                   n_tail_pair), :])

    bpw = b_sz // nw
    n_blk = l_sz

    @functools.partial(
        pl.kernel,
        mesh=mesh,
        compiler_params=pltpu.CompilerParams(needs_layout_passes=False),
        out_type=jax.ShapeDtypeStruct((l_sz, hidden, b_sz), jnp.float32),
        scratch_types=[
            pltpu.VMEM((l_sz, bpw), jnp.int32),
            pltpu.VMEM((l_sz, bpw), jnp.int32),
            pltpu.VMEM((bpw, 128), jnp.float32),
            pltpu.VMEM((bpw, 128), jnp.float32),
            pltpu.VMEM((hidden, bpw), jnp.float32),
            pltpu.VMEM((hidden, bpw), jnp.float32),
            pltpu.SemaphoreType.DMA,
            pltpu.SemaphoreType.DMA,
            pltpu.SemaphoreType.DMA,
            pltpu.SemaphoreType.DMA,
        ],
    )
    def gather_kernel(pairs_hbm, idxt_hbm, out_hbm, idx_v, plist_v,
                      grows0, grows1, oblk0, oblk1,
                      gsem0, gsem1, ssem0, ssem1):
        wid = lax.axis_index("s") * nc + lax.axis_index("c")
        b0 = wid * bpw
        grows = (grows0, grows1)
        oblk = (oblk0, oblk1)
        gsem = (gsem0, gsem1)
        ssem = (ssem0, ssem1)
        rowvec = [_iota16() + (kb * 16) for kb in range(8)]

        pltpu.sync_copy(idxt_hbm.at[:, pl.ds(b0, bpw)], idx_v)

        @plsc.parallel_loop(0, n_blk, unroll=2)
        def prep(i):
            for kb in range(8):
                ids = idx_v[i, pl.ds(kb * 16, 16)]
                plist_v[i, pl.ds(kb * 16, 16)] = lax.shift_right_logical(ids, 1)

        def fire_gather(i, b):
            pltpu.async_copy(pairs_hbm.at[plist_v.at[i]], grows[b], gsem[b])

        def wait_gather(b):
            pltpu.make_async_copy(pairs_hbm.at[pl.ds(0, bpw)],
                                  grows[b], gsem[b]).wait()

        def extract(i, b):
            pcol = []
            for kb in range(8):
                ids = idx_v[i, pl.ds(kb * 16, 16)]
                pcol.append(lax.shift_left(ids & 1, 6))
            src = grows[b]
            dst = oblk[b]

            @plsc.parallel_loop(0, hidden, unroll=8)
            def hrow(h):
                for kb in range(8):
                    val = plsc.load_gather(src, [rowvec[kb], pcol[kb] + h])
                    dst[h, pl.ds(kb * 16, 16)] = val

        def fire_store(i, b):
            pltpu.async_copy(oblk[b], out_hbm.at[i, :, pl.ds(b0, bpw)],
                             ssem[b])

        def wait_store(b):
            pltpu.make_async_copy(oblk[b], out_hbm.at[0, :, pl.ds(b0, bpw)],
                                  ssem[b]).wait()

        def do_block(i, b, first, last):
            if not first:
                wait_store(b)
            if not last:
                fire_gather(i + 1, 1 - b)
            wait_gather(b)
            extract(i, b)
            fire_store(i, b)

        fire_gather(0, 0)
        do_block(0, 0, True, False)
        do_block(1, 1, True, False)

        def step(s, carry):
            do_block(s * 2, 0, False, False)
            do_block(s * 2 + 1, 1, False, False)
            return carry

        lax.fori_loop(1, n_blk // 2 - 1, step, 0)
        do_block(n_blk - 2, 0, False, False)
        do_block(n_blk - 1, 1, False, True)
        wait_store(0)
        wait_store(1)

    return pack_kernel, gather_kernel


def kernel(input_ids, table):
    b_sz, l_sz = input_ids.shape
    vocab, hidden = table.shape
    tt = jnp.transpose(table)
    idx_t = jnp.transpose(input_ids.astype(jnp.int32))
    pack_kernel, gather_kernel = _build(b_sz, l_sz, vocab, hidden)
    nfull = vocab // 128
    tail = jnp.reshape(table[nfull * 128:], ((vocab - nfull * 128) // 2, 128))
    pairs = pack_kernel(tt, tail)
    out_t = gather_kernel(pairs, idx_t)
    return jnp.transpose(out_t, (2, 0, 1))


# speedup vs baseline: 1.1571x; 1.1571x over previous
import functools

import jax
import jax.numpy as jnp
from jax import lax
from jax.experimental import pallas as pl
from jax.experimental.pallas import tpu as pltpu
from jax.experimental.pallas import tpu_sc as plsc


def _iota16():
    return lax.iota(jnp.int32, 16)


@functools.lru_cache(maxsize=None)
def _build(b_sz: int, l_sz: int, vocab: int, hidden: int):
    info = plsc.get_sparse_core_info()
    nc, ns = info.num_cores, info.num_subcores
    nw = nc * ns
    assert hidden == 64 and vocab % 2 == 0 and b_sz % (nw * 128) == 0
    bpw = b_sz // nw
    n_blk = l_sz
    assert n_blk % 2 == 0

    mesh = plsc.VectorSubcoreMesh(core_axis_name="c", subcore_axis_name="s")

    @functools.partial(
        pl.kernel,
        mesh=mesh,
        compiler_params=pltpu.CompilerParams(needs_layout_passes=False),
        out_type=jax.ShapeDtypeStruct((l_sz, hidden, b_sz), jnp.float32),
        scratch_types=[
            pltpu.VMEM((l_sz, bpw), jnp.int32),
            pltpu.VMEM((l_sz, bpw), jnp.int32),
            pltpu.VMEM((bpw, 128), jnp.float32),
            pltpu.VMEM((bpw, 128), jnp.float32),
            pltpu.VMEM((hidden, bpw), jnp.float32),
            pltpu.VMEM((hidden, bpw), jnp.float32),
            pltpu.SemaphoreType.DMA,
            pltpu.SemaphoreType.DMA,
            pltpu.SemaphoreType.DMA,
            pltpu.SemaphoreType.DMA,
        ],
    )
    def gather_kernel(pairs_hbm, idxt_hbm, out_hbm, idx_v, plist_v,
                      grows0, grows1, oblk0, oblk1,
                      gsem0, gsem1, ssem0, ssem1):
        wid = lax.axis_index("s") * nc + lax.axis_index("c")
        b0 = wid * bpw
        grows = (grows0, grows1)
        oblk = (oblk0, oblk1)
        gsem = (gsem0, gsem1)
        ssem = (ssem0, ssem1)
        rowvec = [_iota16() + (kb * 16) for kb in range(8)]

        pltpu.sync_copy(idxt_hbm.at[:, pl.ds(b0, bpw)], idx_v)

        @plsc.parallel_loop(0, n_blk, unroll=2)
        def prep(i):
            for kb in range(8):
                ids = idx_v[i, pl.ds(kb * 16, 16)]
                plist_v[i, pl.ds(kb * 16, 16)] = lax.shift_right_logical(ids, 1)

        def fire_gather(i, b):
            pltpu.async_copy(pairs_hbm.at[plist_v.at[i, pl.ds(0, 64)]],
                             grows[b].at[pl.ds(0, 64), :], gsem[b])
            pltpu.async_copy(pairs_hbm.at[plist_v.at[i, pl.ds(64, 64)]],
                             grows[b].at[pl.ds(64, 64), :], gsem[b])

        def wait_gather(b):
            pltpu.make_async_copy(pairs_hbm.at[pl.ds(0, bpw)],
                                  grows[b], gsem[b]).wait()

        def extract(i, b):
            pcol = []
            for kb in range(8):
                ids = idx_v[i, pl.ds(kb * 16, 16)]
                pcol.append(lax.shift_left(ids & 1, 6))
            src = grows[b]
            dst = oblk[b]

            @plsc.parallel_loop(0, hidden, unroll=8)
            def hrow(h):
                for kb in range(8):
                    val = plsc.load_gather(src, [rowvec[kb], pcol[kb] + h])
                    dst[h, pl.ds(kb * 16, 16)] = val

        def fire_store(i, b):
            pltpu.async_copy(oblk[b], out_hbm.at[i, :, pl.ds(b0, bpw)],
                             ssem[b])

        def wait_store(b):
            pltpu.make_async_copy(oblk[b], out_hbm.at[0, :, pl.ds(b0, bpw)],
                                  ssem[b]).wait()

        def do_block(i, b, first, last):
            if not first:
                wait_store(b)
            if not last:
                fire_gather(i + 1, 1 - b)
            wait_gather(b)
            extract(i, b)
            fire_store(i, b)

        fire_gather(0, 0)
        do_block(0, 0, True, False)
        do_block(1, 1, True, False)

        def step(s, carry):
            do_block(s * 2, 0, False, False)
            do_block(s * 2 + 1, 1, False, False)
            return carry

        lax.fori_loop(1, n_blk // 2 - 1, step, 0)
        do_block(n_blk - 2, 0, False, False)
        do_block(n_blk - 1, 1, False, True)
        wait_store(0)
        wait_store(1)

    return gather_kernel


def kernel(input_ids, table):
    b_sz, l_sz = input_ids.shape
    vocab, hidden = table.shape
    tpair = jnp.reshape(table, (vocab // 2, 2 * hidden))
    idx_t = jnp.transpose(input_ids.astype(jnp.int32))
    fn = _build(b_sz, l_sz, vocab, hidden)
    out_t = fn(tpair, idx_t)
    return jnp.transpose(out_t, (2, 0, 1))


# speedup vs baseline: 1.1784x; 1.0184x over previous
import functools

import jax
import jax.numpy as jnp
from jax import lax
from jax.experimental import pallas as pl
from jax.experimental.pallas import tpu as pltpu
from jax.experimental.pallas import tpu_sc as plsc

IDXW = 128
RPC = 4


@functools.lru_cache(maxsize=None)
def _build(n_tokens: int, vocab: int, hidden: int):
    info = plsc.get_sparse_core_info()
    nc, ns = info.num_cores, info.num_subcores
    nw = nc * ns

    n_rows = n_tokens // IDXW
    rows_per_w = n_rows // nw
    n_chunks = rows_per_w // RPC
    chunk_tokens = RPC * IDXW

    assert n_rows % nw == 0 and rows_per_w % RPC == 0 and n_chunks % 2 == 0

    mesh = plsc.VectorSubcoreMesh(core_axis_name="c", subcore_axis_name="s")

    @functools.partial(
        pl.kernel,
        mesh=mesh,
        compiler_params=pltpu.CompilerParams(use_tc_tiling_on_sc=False),
        out_type=jax.ShapeDtypeStruct((n_tokens, hidden), jnp.float32),
        scratch_types=[
            pltpu.VMEM((rows_per_w, IDXW), jnp.int32),
            pltpu.VMEM((chunk_tokens, hidden), jnp.float32),
            pltpu.VMEM((chunk_tokens, hidden), jnp.float32),
            pltpu.SemaphoreType.DMA,
            pltpu.SemaphoreType.DMA,
            pltpu.SemaphoreType.DMA,
            pltpu.SemaphoreType.DMA,
        ],
    )
    def gather_kernel(table_hbm, idx_hbm, out_hbm, idx_all, rows0, rows1,
                      gsem0, gsem1, ssem0, ssem1):
        wid = lax.axis_index("s") * nc + lax.axis_index("c")
        row0 = wid * rows_per_w
        tok0 = row0 * IDXW

        rows = (rows0, rows1)
        gsem = (gsem0, gsem1)
        ssem = (ssem0, ssem1)

        pltpu.sync_copy(idx_hbm.at[pl.ds(row0, rows_per_w)], idx_all)

        def fire_gathers(i, b):
            for j in range(RPC):
                pltpu.async_copy(
                    table_hbm.at[idx_all.at[i * RPC + j]],
                    rows[b].at[pl.ds(j * IDXW, IDXW)],
                    gsem[b],
                )

        def wait_gathers(b):
            pltpu.make_async_copy(
                out_hbm.at[pl.ds(0, chunk_tokens)], rows[b], gsem[b]
            ).wait()

        def fire_store(i, b):
            pltpu.async_copy(
                rows[b],
                out_hbm.at[pl.ds(tok0 + i * chunk_tokens, chunk_tokens)],
                ssem[b],
            )

        def wait_store(b):
            pltpu.make_async_copy(
                rows[b], out_hbm.at[pl.ds(0, chunk_tokens)], ssem[b]
            ).wait()

        fire_gathers(0, 0)
        fire_gathers(1, 1)
        wait_gathers(0)
        fire_store(0, 0)

        def step(s, carry):
            i0 = s * 2
            wait_store(0)
            fire_gathers(i0, 0)
            wait_gathers(1)
            fire_store(i0 - 1, 1)
            wait_store(1)
            fire_gathers(i0 + 1, 1)
            wait_gathers(0)
            fire_store(i0, 0)
            return carry

        lax.fori_loop(1, n_chunks // 2, step, 0)

        wait_gathers(1)
        fire_store(n_chunks - 1, 1)
        wait_store(0)
        wait_store(1)

    return gather_kernel


def kernel(input_ids, table):
    b, l = input_ids.shape
    vocab, hidden = table.shape
    n_tokens = b * l
    idx = input_ids.reshape(n_tokens // IDXW, IDXW).astype(jnp.int32)
    fn = _build(n_tokens, vocab, hidden)
    out = fn(table, idx)
    return out.reshape(b, l, hidden)
